# Initial kernel scaffold; baseline (speedup 1.0000x reference)
import jax
import jax.numpy as jnp
from jax.experimental import pallas as pl


def kernel(z, Wq, Wv, Wo, ln_w, ln_b, Watt_q, Watt_k, gru_w_ih, gru_w_hh, gru_b_ih, gru_b_hh, Wmf, bmf):
    raise NotImplementedError("write your pallas kernel here")


# speedup vs baseline: 12.7365x; 12.7365x over previous
import jax
import jax.numpy as jnp
from jax.experimental import pallas as pl
from jax.experimental.pallas import tpu as pltpu

_B, _T, _M, _D = 4, 8, 512, 256
_H = 256
_K = 8
_TW = 2
_N = (_TW + 1) * _M

_NEG = -1e30


def _dot_t(a, b):
    return jax.lax.dot_general(a, b, (((1,), (1,)), ((), ())),
                               preferred_element_type=jnp.float32)


def _attn_kernel(zt_ref, zm1_ref, zm2_ref, znt_ref, znm1_ref, znm2_ref,
                 wq_ref, wv_ref, wo_ref,
                 lnw_ref, lnb_ref, wattq_ref, wattk_ref, wih_ref, bih_ref,
                 gi_ref):
    t = pl.program_id(1)
    zt = zt_ref[0, 0]
    z1 = zm1_ref[0, 0]
    z2 = zm2_ref[0, 0]

    qn = znt_ref[0, 0]
    kn = jnp.concatenate([qn, znm1_ref[0, 0], znm2_ref[0, 0]], axis=0)
    kall = jnp.concatenate([zt, z1, z2], axis=0)

    sim = _dot_t(qn, kn)
    col = jax.lax.broadcasted_iota(jnp.int32, (_M, _N), 1)
    nvalid = (jnp.minimum(t, _TW) + 1) * _M
    sim = jnp.where(col < nvalid, sim, _NEG)

    kv = jnp.sum(_dot_t(wattk_ref[...], kall), axis=0, keepdims=True)

    simw = sim
    vals, scores, firsts = [], [], []
    for _ in range(_K):
        mx = jnp.max(simw, axis=1, keepdims=True)
        first = jnp.min(jnp.where(simw == mx, col, _N), axis=1,
                        keepdims=True)
        ohf = col == first
        vals.append(mx)
        scores.append(jnp.sum(jnp.where(ohf, kv, 0.0), axis=1, keepdims=True))
        firsts.append(first)
        simw = jnp.where(ohf, _NEG, simw)
    vals8 = jnp.concatenate(vals, axis=1)
    sc8 = jnp.concatenate(scores, axis=1)

    s = jnp.sum(zt[0:1, :].astype(jnp.bfloat16).astype(jnp.float32)
                * wattq_ref[0:1, :].astype(jnp.bfloat16).astype(jnp.float32))
    att = jax.nn.softmax(s * sc8, axis=1)
    alpha = 0.5 * vals8 + 0.5 * att

    walpha = jnp.zeros((_M, _N), jnp.float32)
    for k in range(_K):
        walpha = walpha + jnp.where(col == firsts[k], alpha[:, k:k + 1], 0.0)
    wsum = jnp.dot(walpha, kall, preferred_element_type=jnp.float32)

    msg = _dot_t(wsum, wv_ref[...])
    xx = msg + _dot_t(zt, wq_ref[...])
    mu = jnp.mean(xx, axis=-1, keepdims=True)
    var = jnp.mean((xx - mu) ** 2, axis=-1, keepdims=True)
    xn = (xx - mu) / jnp.sqrt(var + 1e-5) * lnw_ref[...] + lnb_ref[...]
    inp = _dot_t(xn, wo_ref[...])
    gi_ref[0, 0] = _dot_t(inp, wih_ref[...]) + bih_ref[...]


def _gru_kernel(gi_ref, z_ref, whh_ref, bhh_ref, wmf_ref, bmf_ref,
                h_ref, mem_ref, mem_scr):
    t = pl.program_id(0)

    @pl.when(t == 0)
    def _():
        mem_scr[...] = jnp.zeros_like(mem_scr)

    mem = mem_scr[...]
    gi = gi_ref[...].reshape(_B * _M, 3 * _H)
    gh = _dot_t(mem, whh_ref[...]) + bhh_ref[...]
    i_r, i_z, i_n = gi[:, :_H], gi[:, _H:2 * _H], gi[:, 2 * _H:]
    h_r, h_z, h_n = gh[:, :_H], gh[:, _H:2 * _H], gh[:, 2 * _H:]
    r = jax.nn.sigmoid(i_r + h_r)
    zg = jax.nn.sigmoid(i_z + h_z)
    n = jnp.tanh(i_n + r * h_n)
    mem_new = (1.0 - zg) * n + zg * mem
    mem_scr[...] = mem_new

    z_t = z_ref[...].reshape(_B * _M, _D)
    h = z_t + _dot_t(mem_new, wmf_ref[...]) + bmf_ref[...]
    h_ref[...] = h.reshape(_B, 1, _M, _D)
    mem_ref[...] = mem_new.reshape(_B, _M, _H)


def kernel(z, Wq, Wv, Wo, ln_w, ln_b, Watt_q, Watt_k,
           gru_w_ih, gru_w_hh, gru_b_ih, gru_b_hh, Wmf, bmf):
    nz = jnp.sqrt(jnp.sum(z * z, axis=-1, keepdims=True))
    zn = z / jnp.maximum(nz, 1e-12)

    lnw = ln_w.reshape(1, _D)
    lnb = ln_b.reshape(1, _D)
    bih = gru_b_ih.reshape(1, 3 * _H)
    bhh = gru_b_hh.reshape(1, 3 * _H)
    bmf2 = bmf.reshape(1, _D)

    full = lambda shape: pl.BlockSpec(shape, lambda b, t: (0,) * len(shape))
    gi = pl.pallas_call(
        _attn_kernel,
        grid=(_B, _T),
        in_specs=[
            pl.BlockSpec((1, 1, _M, _D), lambda b, t: (b, t, 0, 0)),
            pl.BlockSpec((1, 1, _M, _D),
                         lambda b, t: (b, jnp.maximum(t - 1, 0), 0, 0)),
            pl.BlockSpec((1, 1, _M, _D),
                         lambda b, t: (b, jnp.maximum(t - 2, 0), 0, 0)),
            pl.BlockSpec((1, 1, _M, _D), lambda b, t: (b, t, 0, 0)),
            pl.BlockSpec((1, 1, _M, _D),
                         lambda b, t: (b, jnp.maximum(t - 1, 0), 0, 0)),
            pl.BlockSpec((1, 1, _M, _D),
                         lambda b, t: (b, jnp.maximum(t - 2, 0), 0, 0)),
            full((_D, _D)), full((_D, _D)), full((_D, _D)),
            full((1, _D)), full((1, _D)),
            full((_D, _D)), full((_D, _D)),
            full((3 * _H, _D)), full((1, 3 * _H)),
        ],
        out_specs=pl.BlockSpec((1, 1, _M, 3 * _H), lambda b, t: (b, t, 0, 0)),
        out_shape=jax.ShapeDtypeStruct((_B, _T, _M, 3 * _H), jnp.float32),
    )(z, z, z, zn, zn, zn, Wq, Wv, Wo, lnw, lnb, Watt_q, Watt_k, gru_w_ih, bih)

    fullt = lambda shape: pl.BlockSpec(shape, lambda t: (0,) * len(shape))
    h, mem = pl.pallas_call(
        _gru_kernel,
        grid=(_T,),
        in_specs=[
            pl.BlockSpec((_B, 1, _M, 3 * _H), lambda t: (0, t, 0, 0)),
            pl.BlockSpec((_B, 1, _M, _D), lambda t: (0, t, 0, 0)),
            fullt((3 * _H, _H)), fullt((1, 3 * _H)),
            fullt((_D, _H)), fullt((1, _D)),
        ],
        out_specs=[
            pl.BlockSpec((_B, 1, _M, _D), lambda t: (0, t, 0, 0)),
            pl.BlockSpec((_B, _M, _H), lambda t: (0, 0, 0)),
        ],
        out_shape=[
            jax.ShapeDtypeStruct((_B, _T, _M, _D), jnp.float32),
            jax.ShapeDtypeStruct((_B, _M, _H), jnp.float32),
        ],
        scratch_shapes=[pltpu.VMEM((_B * _M, _H), jnp.float32)],
    )(gi, z, gru_w_hh, bhh, Wmf, bmf2)
    return h, mem


# speedup vs baseline: 24.7555x; 1.9437x over previous
import jax
import jax.numpy as jnp
from jax.experimental import pallas as pl
from jax.experimental.pallas import tpu as pltpu

_B, _T, _M, _D = 4, 8, 512, 256
_H = 256
_K = 8
_TW = 2
_N = (_TW + 1) * _M

_NEG = -1e30


def _dot_t(a, b):
    return jax.lax.dot_general(a, b, (((1,), (1,)), ((), ())),
                               preferred_element_type=jnp.float32)


def _attn_kernel(zt_ref, zm1_ref, zm2_ref, znt_ref, znm1_ref, znm2_ref,
                 wq_ref, wv_ref, wo_ref,
                 lnw_ref, lnb_ref, wattq_ref, wattk_ref, wih_ref, bih_ref,
                 gi_ref):
    t = pl.program_id(1)
    zt = zt_ref[0, 0]
    z1 = zm1_ref[0, 0]
    z2 = zm2_ref[0, 0]

    qn = znt_ref[0, 0]
    kn = jnp.concatenate([qn, znm1_ref[0, 0], znm2_ref[0, 0]], axis=0)
    kall = jnp.concatenate([zt, z1, z2], axis=0)

    sim = _dot_t(qn, kn)
    col = jax.lax.broadcasted_iota(jnp.int32, (_M, _N), 1)
    nvalid = (jnp.minimum(t, _TW) + 1) * _M
    sim = jnp.where(col < nvalid, sim, _NEG)

    kv = jnp.sum(_dot_t(wattk_ref[...], kall), axis=0, keepdims=True)

    simw = sim
    thr = jnp.max(simw, axis=1, keepdims=True)
    for _ in range(_K - 1):
        simw = jnp.where(sim >= thr, _NEG, sim)
        thr = jnp.max(simw, axis=1, keepdims=True)
    sel = sim >= thr

    s = jnp.sum(zt[0:1, :].astype(jnp.bfloat16).astype(jnp.float32)
                * wattq_ref[0:1, :].astype(jnp.bfloat16).astype(jnp.float32))
    lg = s * kv
    mlg = jnp.max(jnp.where(sel, lg, _NEG), axis=1, keepdims=True)
    e = jnp.where(sel, jnp.exp(lg - mlg), 0.0)
    den = jnp.sum(e, axis=1, keepdims=True)
    walpha = jnp.where(sel, 0.5 * sim, 0.0) + (0.5 / den) * e
    wsum = jnp.dot(walpha, kall, preferred_element_type=jnp.float32)

    msg = _dot_t(wsum, wv_ref[...])
    xx = msg + _dot_t(zt, wq_ref[...])
    mu = jnp.mean(xx, axis=-1, keepdims=True)
    var = jnp.mean((xx - mu) ** 2, axis=-1, keepdims=True)
    xn = (xx - mu) / jnp.sqrt(var + 1e-5) * lnw_ref[...] + lnb_ref[...]
    inp = _dot_t(xn, wo_ref[...])
    gi_ref[0, 0] = _dot_t(inp, wih_ref[...]) + bih_ref[...]


def _gru_kernel(gi_ref, z_ref, whh_ref, bhh_ref, wmf_ref, bmf_ref,
                h_ref, mem_ref, mem_scr):
    t = pl.program_id(0)

    @pl.when(t == 0)
    def _():
        mem_scr[...] = jnp.zeros_like(mem_scr)

    mem = mem_scr[...]
    gi = gi_ref[...].reshape(_B * _M, 3 * _H)
    gh = _dot_t(mem, whh_ref[...]) + bhh_ref[...]
    i_r, i_z, i_n = gi[:, :_H], gi[:, _H:2 * _H], gi[:, 2 * _H:]
    h_r, h_z, h_n = gh[:, :_H], gh[:, _H:2 * _H], gh[:, 2 * _H:]
    r = jax.nn.sigmoid(i_r + h_r)
    zg = jax.nn.sigmoid(i_z + h_z)
    n = jnp.tanh(i_n + r * h_n)
    mem_new = (1.0 - zg) * n + zg * mem
    mem_scr[...] = mem_new

    z_t = z_ref[...].reshape(_B * _M, _D)
    h = z_t + _dot_t(mem_new, wmf_ref[...]) + bmf_ref[...]
    h_ref[...] = h.reshape(_B, 1, _M, _D)
    mem_ref[...] = mem_new.reshape(_B, _M, _H)


def kernel(z, Wq, Wv, Wo, ln_w, ln_b, Watt_q, Watt_k,
           gru_w_ih, gru_w_hh, gru_b_ih, gru_b_hh, Wmf, bmf):
    nz = jnp.sqrt(jnp.sum(z * z, axis=-1, keepdims=True))
    zn = z / jnp.maximum(nz, 1e-12)

    lnw = ln_w.reshape(1, _D)
    lnb = ln_b.reshape(1, _D)
    bih = gru_b_ih.reshape(1, 3 * _H)
    bhh = gru_b_hh.reshape(1, 3 * _H)
    bmf2 = bmf.reshape(1, _D)

    full = lambda shape: pl.BlockSpec(shape, lambda b, t: (0,) * len(shape))
    gi = pl.pallas_call(
        _attn_kernel,
        grid=(_B, _T),
        in_specs=[
            pl.BlockSpec((1, 1, _M, _D), lambda b, t: (b, t, 0, 0)),
            pl.BlockSpec((1, 1, _M, _D),
                         lambda b, t: (b, jnp.maximum(t - 1, 0), 0, 0)),
            pl.BlockSpec((1, 1, _M, _D),
                         lambda b, t: (b, jnp.maximum(t - 2, 0), 0, 0)),
            pl.BlockSpec((1, 1, _M, _D), lambda b, t: (b, t, 0, 0)),
            pl.BlockSpec((1, 1, _M, _D),
                         lambda b, t: (b, jnp.maximum(t - 1, 0), 0, 0)),
            pl.BlockSpec((1, 1, _M, _D),
                         lambda b, t: (b, jnp.maximum(t - 2, 0), 0, 0)),
            full((_D, _D)), full((_D, _D)), full((_D, _D)),
            full((1, _D)), full((1, _D)),
            full((_D, _D)), full((_D, _D)),
            full((3 * _H, _D)), full((1, 3 * _H)),
        ],
        out_specs=pl.BlockSpec((1, 1, _M, 3 * _H), lambda b, t: (b, t, 0, 0)),
        out_shape=jax.ShapeDtypeStruct((_B, _T, _M, 3 * _H), jnp.float32),
    )(z, z, z, zn, zn, zn, Wq, Wv, Wo, lnw, lnb, Watt_q, Watt_k, gru_w_ih, bih)

    fullt = lambda shape: pl.BlockSpec(shape, lambda t: (0,) * len(shape))
    h, mem = pl.pallas_call(
        _gru_kernel,
        grid=(_T,),
        in_specs=[
            pl.BlockSpec((_B, 1, _M, 3 * _H), lambda t: (0, t, 0, 0)),
            pl.BlockSpec((_B, 1, _M, _D), lambda t: (0, t, 0, 0)),
            fullt((3 * _H, _H)), fullt((1, 3 * _H)),
            fullt((_D, _H)), fullt((1, _D)),
        ],
        out_specs=[
            pl.BlockSpec((_B, 1, _M, _D), lambda t: (0, t, 0, 0)),
            pl.BlockSpec((_B, _M, _H), lambda t: (0, 0, 0)),
        ],
        out_shape=[
            jax.ShapeDtypeStruct((_B, _T, _M, _D), jnp.float32),
            jax.ShapeDtypeStruct((_B, _M, _H), jnp.float32),
        ],
        scratch_shapes=[pltpu.VMEM((_B * _M, _H), jnp.float32)],
    )(gi, z, gru_w_hh, bhh, Wmf, bmf2)
    return h, mem


# speedup vs baseline: 25.3141x; 1.0226x over previous
import jax
import jax.numpy as jnp
from jax.experimental import pallas as pl
from jax.experimental.pallas import tpu as pltpu

_B, _T, _M, _D = 4, 8, 512, 256
_H = 256
_K = 8
_TW = 2
_N = (_TW + 1) * _M

_NEG = -1e30


def _dot_t(a, b):
    return jax.lax.dot_general(a, b, (((1,), (1,)), ((), ())),
                               preferred_element_type=jnp.float32)


def _attn_kernel(z_ref, zn_ref,
                 wq_ref, wv_ref, wo_ref,
                 lnw_ref, lnb_ref, wattq_ref, wattk_ref, wih_ref, bih_ref,
                 gi_ref):
    t = pl.program_id(1)
    t1 = jnp.maximum(t - 1, 0)
    t2 = jnp.maximum(t - 2, 0)
    zt = z_ref[0, t]
    z1 = z_ref[0, t1]
    z2 = z_ref[0, t2]

    qn = zn_ref[0, t]
    kn = jnp.concatenate([qn, zn_ref[0, t1], zn_ref[0, t2]], axis=0)
    kall = jnp.concatenate([zt, z1, z2], axis=0)

    sim = _dot_t(qn, kn)
    col = jax.lax.broadcasted_iota(jnp.int32, (_M, _N), 1)
    nvalid = (jnp.minimum(t, _TW) + 1) * _M
    sim = jnp.where(col < nvalid, sim, _NEG)

    kv = jnp.sum(_dot_t(wattk_ref[...], kall), axis=0, keepdims=True)

    simw = sim
    thr = jnp.max(simw, axis=1, keepdims=True)
    for _ in range(_K - 1):
        simw = jnp.where(sim >= thr, _NEG, sim)
        thr = jnp.max(simw, axis=1, keepdims=True)
    sel = sim >= thr

    s = jnp.sum(zt[0:1, :].astype(jnp.bfloat16).astype(jnp.float32)
                * wattq_ref[0:1, :].astype(jnp.bfloat16).astype(jnp.float32))
    lg = s * kv
    mlg = jnp.max(jnp.where(sel, lg, _NEG), axis=1, keepdims=True)
    e = jnp.where(sel, jnp.exp(lg - mlg), 0.0)
    den = jnp.sum(e, axis=1, keepdims=True)
    walpha = jnp.where(sel, 0.5 * sim, 0.0) + (0.5 / den) * e
    wsum = jnp.dot(walpha, kall, preferred_element_type=jnp.float32)

    msg = _dot_t(wsum, wv_ref[...])
    xx = msg + _dot_t(zt, wq_ref[...])
    mu = jnp.mean(xx, axis=-1, keepdims=True)
    var = jnp.mean((xx - mu) ** 2, axis=-1, keepdims=True)
    xn = (xx - mu) / jnp.sqrt(var + 1e-5) * lnw_ref[...] + lnb_ref[...]
    inp = _dot_t(xn, wo_ref[...])
    gi = _dot_t(inp, wih_ref[...]) + bih_ref[...]
    gi_ref[0, 0] = gi.astype(jnp.bfloat16)


def _gru_kernel(gi_ref, z_ref, whh_ref, bhh_ref, wmf_ref, bmf_ref,
                h_ref, mem_ref, mem_scr):
    t = pl.program_id(0)

    @pl.when(t == 0)
    def _():
        mem_scr[...] = jnp.zeros_like(mem_scr)

    mem = mem_scr[...]
    gi = gi_ref[...].reshape(_B * _M, 3 * _H).astype(jnp.float32)
    gh = _dot_t(mem, whh_ref[...]) + bhh_ref[...]
    i_r, i_z, i_n = gi[:, :_H], gi[:, _H:2 * _H], gi[:, 2 * _H:]
    h_r, h_z, h_n = gh[:, :_H], gh[:, _H:2 * _H], gh[:, 2 * _H:]
    r = jax.nn.sigmoid(i_r + h_r)
    zg = jax.nn.sigmoid(i_z + h_z)
    n = jnp.tanh(i_n + r * h_n)
    mem_new = (1.0 - zg) * n + zg * mem
    mem_scr[...] = mem_new

    z_t = z_ref[...].reshape(_B * _M, _D)
    h = z_t + _dot_t(mem_new, wmf_ref[...]) + bmf_ref[...]
    h_ref[...] = h.reshape(_B, 1, _M, _D)
    mem_ref[...] = mem_new.reshape(_B, _M, _H)


def kernel(z, Wq, Wv, Wo, ln_w, ln_b, Watt_q, Watt_k,
           gru_w_ih, gru_w_hh, gru_b_ih, gru_b_hh, Wmf, bmf):
    nz = jnp.sqrt(jnp.sum(z * z, axis=-1, keepdims=True))
    zn = z / jnp.maximum(nz, 1e-12)

    lnw = ln_w.reshape(1, _D)
    lnb = ln_b.reshape(1, _D)
    bih = gru_b_ih.reshape(1, 3 * _H)
    bhh = gru_b_hh.reshape(1, 3 * _H)
    bmf2 = bmf.reshape(1, _D)

    full = lambda shape: pl.BlockSpec(shape, lambda b, t: (0,) * len(shape))
    gi = pl.pallas_call(
        _attn_kernel,
        grid=(_B, _T),
        in_specs=[
            pl.BlockSpec((1, _T, _M, _D), lambda b, t: (b, 0, 0, 0)),
            pl.BlockSpec((1, _T, _M, _D), lambda b, t: (b, 0, 0, 0)),
            full((_D, _D)), full((_D, _D)), full((_D, _D)),
            full((1, _D)), full((1, _D)),
            full((_D, _D)), full((_D, _D)),
            full((3 * _H, _D)), full((1, 3 * _H)),
        ],
        out_specs=pl.BlockSpec((1, 1, _M, 3 * _H), lambda b, t: (b, t, 0, 0)),
        out_shape=jax.ShapeDtypeStruct((_B, _T, _M, 3 * _H), jnp.bfloat16),
    )(z, zn, Wq, Wv, Wo, lnw, lnb, Watt_q, Watt_k, gru_w_ih, bih)

    fullt = lambda shape: pl.BlockSpec(shape, lambda t: (0,) * len(shape))
    h, mem = pl.pallas_call(
        _gru_kernel,
        grid=(_T,),
        in_specs=[
            pl.BlockSpec((_B, 1, _M, 3 * _H), lambda t: (0, t, 0, 0)),
            pl.BlockSpec((_B, 1, _M, _D), lambda t: (0, t, 0, 0)),
            fullt((3 * _H, _H)), fullt((1, 3 * _H)),
            fullt((_D, _H)), fullt((1, _D)),
        ],
        out_specs=[
            pl.BlockSpec((_B, 1, _M, _D), lambda t: (0, t, 0, 0)),
            pl.BlockSpec((_B, _M, _H), lambda t: (0, 0, 0)),
        ],
        out_shape=[
            jax.ShapeDtypeStruct((_B, _T, _M, _D), jnp.float32),
            jax.ShapeDtypeStruct((_B, _M, _H), jnp.float32),
        ],
        scratch_shapes=[pltpu.VMEM((_B * _M, _H), jnp.float32)],
    )(gi, z, gru_w_hh, bhh, Wmf, bmf2)
    return h, mem


# speedup vs baseline: 25.7588x; 1.0176x over previous
import jax
import jax.numpy as jnp
from jax.experimental import pallas as pl
from jax.experimental.pallas import tpu as pltpu

_B, _T, _M, _D = 4, 8, 512, 256
_H = 256
_K = 8
_TW = 2
_N = (_TW + 1) * _M

_NEG = -1e30


def _dot_t(a, b):
    return jax.lax.dot_general(a, b, (((1,), (1,)), ((), ())),
                               preferred_element_type=jnp.float32)


def _attn_kernel(z_ref, zn_ref,
                 wq_ref, wv_ref, wo_ref,
                 lnw_ref, lnb_ref, wattq_ref, wattk_ref, wih_ref, bih_ref,
                 gi_ref):
    t = pl.program_id(1)
    t1 = jnp.maximum(t - 1, 0)
    t2 = jnp.maximum(t - 2, 0)
    zt = z_ref[0, t]
    z1 = z_ref[0, t1]
    z2 = z_ref[0, t2]

    qn = zn_ref[0, t]
    kn = jnp.concatenate([qn, zn_ref[0, t1], zn_ref[0, t2]], axis=0)
    kall = jnp.concatenate([zt, z1, z2], axis=0)

    sim = _dot_t(qn, kn)
    col = jax.lax.broadcasted_iota(jnp.int32, (_M, _N), 1)
    nvalid = (jnp.minimum(t, _TW) + 1) * _M
    sim = jnp.where(col < nvalid, sim, _NEG)

    kv = jnp.sum(_dot_t(wattk_ref[...], kall), axis=0, keepdims=True)

    a = sim[:, :_N // 2]
    b = sim[:, _N // 2:]
    pool = jnp.maximum(a, b)
    sub = jnp.minimum(a, b)
    thr = jnp.max(pool, axis=1, keepdims=True)
    for _ in range(_K - 1):
        hit = pool == thr
        pool = jnp.where(hit, sub, pool)
        sub = jnp.where(hit, _NEG, sub)
        thr = jnp.max(pool, axis=1, keepdims=True)
    sel = sim >= thr

    s = jnp.sum(zt[0:1, :].astype(jnp.bfloat16).astype(jnp.float32)
                * wattq_ref[0:1, :].astype(jnp.bfloat16).astype(jnp.float32))
    lg = s * kv
    mlg = jnp.max(jnp.where(sel, lg, _NEG), axis=1, keepdims=True)
    e = jnp.where(sel, jnp.exp(lg - mlg), 0.0)
    den = jnp.sum(e, axis=1, keepdims=True)
    walpha = jnp.where(sel, 0.5 * sim, 0.0) + (0.5 / den) * e
    wsum = jnp.dot(walpha, kall, preferred_element_type=jnp.float32)

    msg = _dot_t(wsum, wv_ref[...])
    xx = msg + _dot_t(zt, wq_ref[...])
    mu = jnp.mean(xx, axis=-1, keepdims=True)
    var = jnp.mean((xx - mu) ** 2, axis=-1, keepdims=True)
    xn = (xx - mu) / jnp.sqrt(var + 1e-5) * lnw_ref[...] + lnb_ref[...]
    inp = _dot_t(xn, wo_ref[...])
    gi = _dot_t(inp, wih_ref[...]) + bih_ref[...]
    gi_ref[0, 0] = gi.astype(jnp.bfloat16)


def _gru_kernel(gi_ref, z_ref, whh_ref, bhh_ref, wmf_ref, bmf_ref,
                h_ref, mem_ref, mem_scr):
    t = pl.program_id(0)

    @pl.when(t == 0)
    def _():
        mem_scr[...] = jnp.zeros_like(mem_scr)

    mem = mem_scr[...]
    gi = gi_ref[...].reshape(_B * _M, 3 * _H).astype(jnp.float32)
    gh = _dot_t(mem, whh_ref[...]) + bhh_ref[...]
    i_r, i_z, i_n = gi[:, :_H], gi[:, _H:2 * _H], gi[:, 2 * _H:]
    h_r, h_z, h_n = gh[:, :_H], gh[:, _H:2 * _H], gh[:, 2 * _H:]
    r = jax.nn.sigmoid(i_r + h_r)
    zg = jax.nn.sigmoid(i_z + h_z)
    n = jnp.tanh(i_n + r * h_n)
    mem_new = (1.0 - zg) * n + zg * mem
    mem_scr[...] = mem_new

    z_t = z_ref[...].reshape(_B * _M, _D)
    h = z_t + _dot_t(mem_new, wmf_ref[...]) + bmf_ref[...]
    h_ref[...] = h.reshape(_B, 1, _M, _D)
    mem_ref[...] = mem_new.reshape(_B, _M, _H)


def kernel(z, Wq, Wv, Wo, ln_w, ln_b, Watt_q, Watt_k,
           gru_w_ih, gru_w_hh, gru_b_ih, gru_b_hh, Wmf, bmf):
    nz = jnp.sqrt(jnp.sum(z * z, axis=-1, keepdims=True))
    zn = z / jnp.maximum(nz, 1e-12)

    lnw = ln_w.reshape(1, _D)
    lnb = ln_b.reshape(1, _D)
    bih = gru_b_ih.reshape(1, 3 * _H)
    bhh = gru_b_hh.reshape(1, 3 * _H)
    bmf2 = bmf.reshape(1, _D)

    full = lambda shape: pl.BlockSpec(shape, lambda b, t: (0,) * len(shape))
    gi = pl.pallas_call(
        _attn_kernel,
        grid=(_B, _T),
        in_specs=[
            pl.BlockSpec((1, _T, _M, _D), lambda b, t: (b, 0, 0, 0)),
            pl.BlockSpec((1, _T, _M, _D), lambda b, t: (b, 0, 0, 0)),
            full((_D, _D)), full((_D, _D)), full((_D, _D)),
            full((1, _D)), full((1, _D)),
            full((_D, _D)), full((_D, _D)),
            full((3 * _H, _D)), full((1, 3 * _H)),
        ],
        out_specs=pl.BlockSpec((1, 1, _M, 3 * _H), lambda b, t: (b, t, 0, 0)),
        out_shape=jax.ShapeDtypeStruct((_B, _T, _M, 3 * _H), jnp.bfloat16),
    )(z, zn, Wq, Wv, Wo, lnw, lnb, Watt_q, Watt_k, gru_w_ih, bih)

    fullt = lambda shape: pl.BlockSpec(shape, lambda t: (0,) * len(shape))
    h, mem = pl.pallas_call(
        _gru_kernel,
        grid=(_T,),
        in_specs=[
            pl.BlockSpec((_B, 1, _M, 3 * _H), lambda t: (0, t, 0, 0)),
            pl.BlockSpec((_B, 1, _M, _D), lambda t: (0, t, 0, 0)),
            fullt((3 * _H, _H)), fullt((1, 3 * _H)),
            fullt((_D, _H)), fullt((1, _D)),
        ],
        out_specs=[
            pl.BlockSpec((_B, 1, _M, _D), lambda t: (0, t, 0, 0)),
            pl.BlockSpec((_B, _M, _H), lambda t: (0, 0, 0)),
        ],
        out_shape=[
            jax.ShapeDtypeStruct((_B, _T, _M, _D), jnp.float32),
            jax.ShapeDtypeStruct((_B, _M, _H), jnp.float32),
        ],
        scratch_shapes=[pltpu.VMEM((_B * _M, _H), jnp.float32)],
    )(gi, z, gru_w_hh, bhh, Wmf, bmf2)
    return h, mem
